# baseline (speedup 1.0000x reference)
import functools

import jax
import jax.numpy as jnp
from jax import lax
from jax.experimental import pallas as pl
from jax.experimental.pallas import tpu as pltpu
from jax.experimental.pallas import tpu_sc as plsc

N = 10000
E = 320000
D = 128
H = 8
F = 16
HF = H * F
TBL = HF + 2 * H
NPAD = 10240
RB = 1280
NTC = NPAD // RB
NS = 16
C = 64
IBLK = 32
NIB = 10
KCH = NIB * IBLK
EPW = KCH * C
EPAD = NS * EPW
RPS = NPAD // NS

_HIGH = jax.lax.Precision.HIGHEST


def _dot(a, b):
    return jnp.dot(a, b, precision=_HIGH, preferred_element_type=jnp.float32)


def _prologue_body(x_ref, w0_ref, a0_ref, w1_ref, a1_ref,
                   t0_ref, s0_ref, t1_ref, s1_ref):
    xb = x_ref[...]
    z8 = jnp.zeros((RB, H), jnp.float32)
    for w_ref, a_ref, t_ref, s_ref in ((w0_ref, a0_ref, t0_ref, s0_ref),
                                       (w1_ref, a1_ref, t1_ref, s1_ref)):
        feat = _dot(xb, w_ref[...])
        elr = _dot(feat, a_ref[...])
        t_ref[...] = jnp.concatenate([feat, elr[:, :H], z8], axis=1)
        s_ref[...] = jnp.concatenate([elr[:, H:], z8], axis=1)


def _prologue(x_pad, W0, A0, W1, A1):
    full = lambda s: pl.BlockSpec(s, lambda i: (0, 0))
    return pl.pallas_call(
        _prologue_body,
        grid=(NTC,),
        in_specs=[
            pl.BlockSpec((RB, D), lambda i: (i, 0)),
            full((D, HF)), full((D, 2 * H)),
            full((D, HF)), full((D, 2 * H)),
        ],
        out_specs=[
            pl.BlockSpec((RB, TBL), lambda i: (i, 0)),
            pl.BlockSpec((RB, 16), lambda i: (i, 0)),
            pl.BlockSpec((RB, TBL), lambda i: (i, 0)),
            pl.BlockSpec((RB, 16), lambda i: (i, 0)),
        ],
        out_shape=[
            jax.ShapeDtypeStruct((NPAD, TBL), jnp.float32),
            jax.ShapeDtypeStruct((NPAD, 16), jnp.float32),
            jax.ShapeDtypeStruct((NPAD, TBL), jnp.float32),
            jax.ShapeDtypeStruct((NPAD, 16), jnp.float32),
        ],
    )(x_pad, W0, A0, W1, A1)


def _bcast16(v, h):
    idx = jnp.full((16, 1), h, dtype=jnp.int32)
    dn = lax.GatherDimensionNumbers(
        offset_dims=(), collapsed_slice_dims=(0,), start_index_map=(0,))
    return lax.gather(v, idx, dn, slice_sizes=(1,),
                      mode=lax.GatherScatterMode.PROMISE_IN_BOUNDS)


def _edge_chunk(rows_v, erd_v):
    @plsc.parallel_loop(0, C, unroll=4)
    def _edge(i):
        v = rows_v[i, pl.ds(HF, 16)]
        w = erd_v[i, :]
        e = v + w
        e = jnp.maximum(e, 0.2 * e)
        p = jnp.exp(e)
        rows_v[i, pl.ds(HF, 16)] = p
        for h in range(H):
            ph = _bcast16(p, h)
            blk = rows_v[i, pl.ds(16 * h, 16)]
            rows_v[i, pl.ds(16 * h, 16)] = blk * ph


def _run_metapath(table_hbm, small_hbm, src_hbm, dst_hbm, sid,
                  srcs_v, dsts_v, rows_v, erd_v, sems, isems, acc):
    def _issue(b, sl, j):
        pltpu.async_copy(table_hbm.at[srcs_v.at[sl, j]], rows_v.at[b],
                         sems[2 * b])
        pltpu.async_copy(small_hbm.at[dsts_v.at[sl, j]], erd_v.at[b],
                         sems[2 * b + 1])

    def _await(b):
        pltpu.make_async_copy(table_hbm.at[srcs_v.at[0, 0]], rows_v.at[b],
                              sems[2 * b]).wait()
        pltpu.make_async_copy(small_hbm.at[dsts_v.at[0, 0]], erd_v.at[b],
                              sems[2 * b + 1]).wait()

    pltpu.sync_copy(src_hbm.at[sid, 0], srcs_v.at[0])
    pltpu.sync_copy(dst_hbm.at[sid, 0], dsts_v.at[0])

    for ib in range(NIB):
        sl = ib % 2
        if ib + 1 < NIB:
            pltpu.async_copy(src_hbm.at[sid, ib + 1], srcs_v.at[1 - sl],
                             isems[0])
            pltpu.async_copy(dst_hbm.at[sid, ib + 1], dsts_v.at[1 - sl],
                             isems[1])
        _issue(0, sl, 0)
        _issue(1, sl, 1)

        @pl.loop(0, IBLK, step=2)
        def _chunk(l):
            for b in range(2):
                ll = l + b
                _await(b)
                _edge_chunk(rows_v.at[b], erd_v.at[b])
                pltpu.sync_copy(rows_v.at[b], acc.at[dsts_v.at[sl, ll]],
                                add=True)

                @pl.when(ll + 2 < IBLK)
                def _():
                    _issue(b, sl, ll + 2)

        if ib + 1 < NIB:
            pltpu.make_async_copy(src_hbm.at[sid, 0], srcs_v.at[0],
                                  isems[0]).wait()
            pltpu.make_async_copy(dst_hbm.at[sid, 0], dsts_v.at[0],
                                  isems[1]).wait()


def _sc_edge_body(t0_hbm, s0_hbm, t1_hbm, s1_hbm, src_hbm, dst_hbm,
                  zeros_hbm, out_hbm,
                  srcs_v, dsts_v, rows_v, erd_v,
                  sem0, sem1, sem2, sem3, isem0, isem1, acc):
    cid = lax.axis_index("c")
    sid = lax.axis_index("s")
    r0 = sid * RPS
    pltpu.sync_copy(zeros_hbm.at[pl.ds(r0, RPS)], acc.at[pl.ds(r0, RPS)])
    plsc.subcore_barrier()

    sems = (sem0, sem1, sem2, sem3)
    isems = (isem0, isem1)

    @pl.when(cid == 0)
    def _():
        _run_metapath(t0_hbm, s0_hbm, src_hbm.at[0], dst_hbm.at[0], sid,
                      srcs_v, dsts_v, rows_v, erd_v, sems, isems, acc)

    @pl.when(cid == 1)
    def _():
        _run_metapath(t1_hbm, s1_hbm, src_hbm.at[1], dst_hbm.at[1], sid,
                      srcs_v, dsts_v, rows_v, erd_v, sems, isems, acc)

    plsc.subcore_barrier()
    pltpu.sync_copy(acc.at[pl.ds(r0, RPS)], out_hbm.at[cid, pl.ds(r0, RPS)])


_sc_edge_pass = pl.kernel(
    _sc_edge_body,
    out_type=jax.ShapeDtypeStruct((2, NPAD, TBL), jnp.float32),
    mesh=plsc.VectorSubcoreMesh(core_axis_name="c", subcore_axis_name="s"),
    compiler_params=pltpu.CompilerParams(use_tc_tiling_on_sc=False),
    scratch_types=[
        pltpu.VMEM((2, IBLK, C), jnp.int32),
        pltpu.VMEM((2, IBLK, C), jnp.int32),
        pltpu.VMEM((2, C, TBL), jnp.float32),
        pltpu.VMEM((2, C, 16), jnp.float32),
        pltpu.SemaphoreType.DMA,
        pltpu.SemaphoreType.DMA,
        pltpu.SemaphoreType.DMA,
        pltpu.SemaphoreType.DMA,
        pltpu.SemaphoreType.DMA,
        pltpu.SemaphoreType.DMA,
        pltpu.VMEM_SHARED((NPAD, TBL), jnp.float32),
    ],
)


def _elu(x):
    return jnp.where(x > 0, x, jnp.exp(jnp.minimum(x, 0.0)) - 1.0)


def _merge_body(p_ref, b0_ref, b1_ref, ws1_ref, bs1_ref, ws2_ref,
                brd_ref, z0_ref, z1_ref, sums_ref):
    i = pl.program_id(0)
    brd = brd_ref[...]
    zs = []
    for k, b_ref in ((0, b0_ref), (1, b1_ref)):
        m = p_ref[k]
        num = m[:, :HF]
        den = m[:, HF:HF + H]
        rec = 1.0 / (den + 1e-9)
        recb = _dot(rec, brd)
        zs.append(_elu(num * recb + b_ref[...]))
    z0_ref[...] = zs[0]
    z1_ref[...] = zs[1]
    rows = i * RB + lax.broadcasted_iota(jnp.int32, (RB, 1), 0)
    mask = rows < N
    lane = lax.broadcasted_iota(jnp.int32, (1, HF), 1)
    acc = jnp.zeros((1, HF), jnp.float32)
    for k, z in enumerate(zs):
        t = _dot(jnp.tanh(_dot(z, ws1_ref[...]) + bs1_ref[...]), ws2_ref[...])
        s = jnp.sum(jnp.where(mask, t, 0.0))
        acc = acc + jnp.where(lane == k, s, 0.0)
    sums_ref[pl.ds(i, 1), :] = acc


def _merge(parts, b0, b1, Ws1, bs1, Ws2, Brd):
    full = lambda s: pl.BlockSpec(s, lambda i: (0, 0))
    return pl.pallas_call(
        _merge_body,
        grid=(NTC,),
        in_specs=[
            pl.BlockSpec((2, RB, TBL), lambda i: (0, i, 0)),
            full((1, HF)), full((1, HF)),
            full((HF, HF)), full((1, HF)), full((HF, 1)),
            full((H, HF)),
        ],
        out_specs=[
            pl.BlockSpec((RB, HF), lambda i: (i, 0)),
            pl.BlockSpec((RB, HF), lambda i: (i, 0)),
            pl.BlockSpec((NTC, HF), lambda i: (0, 0)),
        ],
        out_shape=[
            jax.ShapeDtypeStruct((NPAD, HF), jnp.float32),
            jax.ShapeDtypeStruct((NPAD, HF), jnp.float32),
            jax.ShapeDtypeStruct((NTC, HF), jnp.float32),
        ],
    )(parts, b0, b1, Ws1, bs1, Ws2, Brd)


def _final_body(z0_ref, z1_ref, sums_ref, o_ref):
    s = sums_ref[...]
    w0 = jnp.sum(s[:, 0]) / N
    w1 = jnp.sum(s[:, 1]) / N
    m = jnp.maximum(w0, w1)
    e0 = jnp.exp(w0 - m)
    e1 = jnp.exp(w1 - m)
    beta0 = e0 / (e0 + e1)
    beta1 = e1 / (e0 + e1)
    o_ref[...] = beta0 * z0_ref[...] + beta1 * z1_ref[...]


def _final(z0, z1, sums):
    return pl.pallas_call(
        _final_body,
        grid=(NTC,),
        in_specs=[
            pl.BlockSpec((RB, HF), lambda i: (i, 0)),
            pl.BlockSpec((RB, HF), lambda i: (i, 0)),
            pl.BlockSpec((NTC, HF), lambda i: (0, 0)),
        ],
        out_specs=pl.BlockSpec((RB, HF), lambda i: (i, 0)),
        out_shape=jax.ShapeDtypeStruct((NPAD, HF), jnp.float32),
    )(z0, z1, sums)


def _attn_mat(attn_l, attn_r):
    eye = jnp.eye(H, dtype=jnp.float32)
    al = (attn_l[:, :, None] * eye[:, None, :]).reshape(HF, H)
    ar = (attn_r[:, :, None] * eye[:, None, :]).reshape(HF, H)
    return jnp.concatenate([al, ar], axis=1)


def _pad_edges(ei):
    src = jnp.concatenate(
        [ei[0], jnp.full((EPAD - E,), N, jnp.int32)]).reshape(NS, NIB, IBLK, C)
    dst = jnp.concatenate(
        [ei[1],
         jnp.full((EPAD - E,), NPAD - 1, jnp.int32)]).reshape(NS, NIB, IBLK, C)
    return src, dst


def kernel(x, edge_index_0, edge_index_1, W_g0, attn_l0, attn_r0, bias0,
           W_g1, attn_l1, attn_r1, bias1, W_s1, b_s1, W_s2):
    x_pad = jnp.zeros((NPAD, D), jnp.float32).at[:N].set(x)
    A0 = _attn_mat(attn_l0, attn_r0)
    A1 = _attn_mat(attn_l1, attn_r1)
    eye = jnp.eye(H, dtype=jnp.float32)
    Brd = (eye[:, :, None] * jnp.ones((1, 1, F), jnp.float32)).reshape(H, HF)
    zeros_tbl = jnp.zeros((NPAD, TBL), jnp.float32)

    table0, small0, table1, small1 = _prologue(x_pad, W_g0, A0, W_g1, A1)

    src0, dst0 = _pad_edges(edge_index_0)
    src1, dst1 = _pad_edges(edge_index_1)
    src = jnp.stack([src0, src1])
    dst = jnp.stack([dst0, dst1])
    parts = _sc_edge_pass(table0, small0, table1, small1, src, dst, zeros_tbl)

    z0, z1, sums = _merge(parts,
                          bias0.reshape(1, HF), bias1.reshape(1, HF),
                          W_s1, b_s1.reshape(1, HF), W_s2, Brd)
    out = _final(z0, z1, sums)
    return out[:N]


# speedup vs baseline: 1.3227x; 1.3227x over previous
import functools

import jax
import jax.numpy as jnp
from jax import lax
from jax.experimental import pallas as pl
from jax.experimental.pallas import tpu as pltpu
from jax.experimental.pallas import tpu_sc as plsc

N = 10000
E = 320000
D = 128
H = 8
F = 16
HF = H * F
TBL = HF + 2 * H
NPAD = 10240
RB = 1280
NTC = NPAD // RB
NS = 16
C = 64
IBLK = 20
NIB = 16
KCH = NIB * IBLK
EPW = KCH * C
EPAD = NS * EPW
RPS = NPAD // NS

_HIGH = jax.lax.Precision.HIGHEST


def _dot(a, b):
    return jnp.dot(a, b, precision=_HIGH, preferred_element_type=jnp.float32)


def _prologue_body(x_ref, p_ref, w0_ref, a0_ref, w1_ref, a1_ref,
                   f0_ref, el0_ref, er0_ref, f1_ref, el1_ref, er1_ref):
    xb = x_ref[...]
    perm = p_ref[...]
    z8 = jnp.zeros((RB, H), jnp.float32)
    for w_ref, a_ref, f_ref, el_ref, er_ref in (
            (w0_ref, a0_ref, f0_ref, el0_ref, er0_ref),
            (w1_ref, a1_ref, f1_ref, el1_ref, er1_ref)):
        feat = _dot(xb, w_ref[...])
        elr = _dot(feat, a_ref[...])
        f_ref[...] = _dot(feat, perm).astype(jnp.bfloat16)
        el_ref[...] = jnp.concatenate([elr[:, :H], z8], axis=1)
        er_ref[...] = jnp.concatenate([elr[:, H:], z8], axis=1)


def _prologue(x_pad, P, W0, A0, W1, A1):
    full = lambda s: pl.BlockSpec(s, lambda i: (0, 0))
    return pl.pallas_call(
        _prologue_body,
        grid=(NTC,),
        in_specs=[
            pl.BlockSpec((RB, D), lambda i: (i, 0)),
            full((HF, HF)),
            full((D, HF)), full((D, 2 * H)),
            full((D, HF)), full((D, 2 * H)),
        ],
        out_specs=[
            pl.BlockSpec((RB, HF), lambda i: (i, 0)),
            pl.BlockSpec((RB, 16), lambda i: (i, 0)),
            pl.BlockSpec((RB, 16), lambda i: (i, 0)),
            pl.BlockSpec((RB, HF), lambda i: (i, 0)),
            pl.BlockSpec((RB, 16), lambda i: (i, 0)),
            pl.BlockSpec((RB, 16), lambda i: (i, 0)),
        ],
        out_shape=[
            jax.ShapeDtypeStruct((NPAD, HF), jnp.bfloat16),
            jax.ShapeDtypeStruct((NPAD, 16), jnp.float32),
            jax.ShapeDtypeStruct((NPAD, 16), jnp.float32),
            jax.ShapeDtypeStruct((NPAD, HF), jnp.bfloat16),
            jax.ShapeDtypeStruct((NPAD, 16), jnp.float32),
            jax.ShapeDtypeStruct((NPAD, 16), jnp.float32),
        ],
    )(x_pad, P, W0, A0, W1, A1)


def _bcast16(v, h):
    idx = jnp.full((16, 1), h, dtype=jnp.int32)
    dn = lax.GatherDimensionNumbers(
        offset_dims=(), collapsed_slice_dims=(0,), start_index_map=(0,))
    return lax.gather(v, idx, dn, slice_sizes=(1,),
                      mode=lax.GatherScatterMode.PROMISE_IN_BOUNDS)


def _edge_chunk(fb_v, els_v, erd_v, out_v):
    @plsc.parallel_loop(0, C, unroll=2)
    def _edge(i):
        v = els_v[i, :]
        w = erd_v[i, :]
        e = v + w
        e = jnp.maximum(e, 0.2 * e)
        p = jnp.exp(e)
        out_v[i, pl.ds(HF, 16)] = p
        for g in range(4):
            fb = fb_v[i, pl.ds(32 * g, 32)]
            a, b = plsc.unpack(fb, format=plsc.PackFormat.INTERLEAVED,
                               preferred_element_type=jnp.float32)
            out_v[i, pl.ds(32 * g, 16)] = a * _bcast16(p, 2 * g)
            out_v[i, pl.ds(32 * g + 16, 16)] = b * _bcast16(p, 2 * g + 1)


def _run_metapath(feat_hbm, el_hbm, er_hbm, src_hbm, dst_hbm, zeros_hbm, sid,
                  srcs_v, dsts_v, fb_v, els_v, erd_v, out_v,
                  gsems, ssems, isems, acc):
    def _issue(b, sl, dsl, j):
        pltpu.async_copy(feat_hbm.at[srcs_v.at[sl, j]], fb_v.at[b],
                         gsems[3 * b])
        pltpu.async_copy(el_hbm.at[srcs_v.at[sl, j]], els_v.at[b],
                         gsems[3 * b + 1])
        pltpu.async_copy(er_hbm.at[dsts_v.at[dsl, j]], erd_v.at[b],
                         gsems[3 * b + 2])

    def _await_gather(b):
        pltpu.make_async_copy(feat_hbm.at[srcs_v.at[0, 0]], fb_v.at[b],
                              gsems[3 * b]).wait()
        pltpu.make_async_copy(el_hbm.at[srcs_v.at[0, 0]], els_v.at[b],
                              gsems[3 * b + 1]).wait()
        pltpu.make_async_copy(er_hbm.at[dsts_v.at[0, 0]], erd_v.at[b],
                              gsems[3 * b + 2]).wait()

    def _issue_scatter(b, dsl, j):
        pltpu.async_copy(out_v.at[b], acc.at[dsts_v.at[dsl, j]], ssems[b],
                         add=True)

    def _await_scatter(b):
        pltpu.make_async_copy(out_v.at[b], acc.at[dsts_v.at[0, 0]],
                              ssems[b]).wait()

    pltpu.sync_copy(src_hbm.at[sid, 0], srcs_v.at[0])
    pltpu.sync_copy(dst_hbm.at[sid, 0], dsts_v.at[0])

    for b in range(2):
        pltpu.sync_copy(zeros_hbm.at[pl.ds(0, C)], out_v.at[b])
        _issue_scatter(b, 0, b)
    _issue(0, 0, 0, 0)
    _issue(1, 0, 0, 1)

    @pl.loop(0, NIB)
    def _block(ib):
        sl = ib % 2
        dsl = ib % 3
        nsl = (ib + 1) % 2
        ndsl = (ib + 1) % 3

        @pl.when(ib + 1 < NIB)
        def _():
            pltpu.async_copy(src_hbm.at[sid, ib + 1], srcs_v.at[nsl],
                             isems[0])
            pltpu.async_copy(dst_hbm.at[sid, ib + 1], dsts_v.at[ndsl],
                             isems[1])

        @pl.loop(0, IBLK, step=2)
        def _chunk(l):
            for b in range(2):
                ll = l + b
                _await_gather(b)
                _await_scatter(b)
                _edge_chunk(fb_v.at[b], els_v.at[b], erd_v.at[b], out_v.at[b])
                _issue_scatter(b, dsl, ll)

                @pl.when(ll + 2 < IBLK)
                def _():
                    _issue(b, sl, dsl, ll + 2)

        @pl.when(ib + 1 < NIB)
        def _():
            pltpu.make_async_copy(src_hbm.at[sid, 0], srcs_v.at[0],
                                  isems[0]).wait()
            pltpu.make_async_copy(dst_hbm.at[sid, 0], dsts_v.at[0],
                                  isems[1]).wait()
            _issue(0, nsl, ndsl, 0)
            _issue(1, nsl, ndsl, 1)

    _await_scatter(0)
    _await_scatter(1)


def _sc_edge_body(f0_hbm, el0_hbm, er0_hbm, f1_hbm, el1_hbm, er1_hbm,
                  src_hbm, dst_hbm, zeros_hbm, out_hbm,
                  srcs_v, dsts_v, fb_v, els_v, erd_v, out_v,
                  g0, g1, g2, g3, g4, g5, s0, s1, i0, i1, acc):
    cid = lax.axis_index("c")
    sid = lax.axis_index("s")
    r0 = sid * RPS
    pltpu.sync_copy(zeros_hbm.at[pl.ds(r0, RPS)], acc.at[pl.ds(r0, RPS)])
    plsc.subcore_barrier()

    gsems = (g0, g1, g2, g3, g4, g5)
    ssems = (s0, s1)
    isems = (i0, i1)

    @pl.when(cid == 0)
    def _():
        _run_metapath(f0_hbm, el0_hbm, er0_hbm, src_hbm.at[0], dst_hbm.at[0],
                      zeros_hbm, sid, srcs_v, dsts_v, fb_v, els_v, erd_v,
                      out_v, gsems, ssems, isems, acc)

    @pl.when(cid == 1)
    def _():
        _run_metapath(f1_hbm, el1_hbm, er1_hbm, src_hbm.at[1], dst_hbm.at[1],
                      zeros_hbm, sid, srcs_v, dsts_v, fb_v, els_v, erd_v,
                      out_v, gsems, ssems, isems, acc)

    plsc.subcore_barrier()
    pltpu.sync_copy(acc.at[pl.ds(r0, RPS)], out_hbm.at[cid, pl.ds(r0, RPS)])


_sc_edge_pass = pl.kernel(
    _sc_edge_body,
    out_type=jax.ShapeDtypeStruct((2, NPAD, TBL), jnp.float32),
    mesh=plsc.VectorSubcoreMesh(core_axis_name="c", subcore_axis_name="s"),
    compiler_params=pltpu.CompilerParams(use_tc_tiling_on_sc=False,
                                         needs_layout_passes=False),
    scratch_types=[
        pltpu.VMEM((2, IBLK, C), jnp.int32),
        pltpu.VMEM((3, IBLK, C), jnp.int32),
        pltpu.VMEM((2, C, HF), jnp.bfloat16),
        pltpu.VMEM((2, C, 16), jnp.float32),
        pltpu.VMEM((2, C, 16), jnp.float32),
        pltpu.VMEM((2, C, TBL), jnp.float32),
        pltpu.SemaphoreType.DMA,
        pltpu.SemaphoreType.DMA,
        pltpu.SemaphoreType.DMA,
        pltpu.SemaphoreType.DMA,
        pltpu.SemaphoreType.DMA,
        pltpu.SemaphoreType.DMA,
        pltpu.SemaphoreType.DMA,
        pltpu.SemaphoreType.DMA,
        pltpu.SemaphoreType.DMA,
        pltpu.SemaphoreType.DMA,
        pltpu.VMEM_SHARED((NPAD, TBL), jnp.float32),
    ],
)


def _elu(x):
    return jnp.where(x > 0, x, jnp.exp(jnp.minimum(x, 0.0)) - 1.0)


def _merge_body(p_ref, b0_ref, b1_ref, ws1_ref, bs1_ref, ws2_ref,
                brd_ref, z0_ref, z1_ref, sums_ref):
    i = pl.program_id(0)
    brd = brd_ref[...]
    zs = []
    for k, b_ref in ((0, b0_ref), (1, b1_ref)):
        m = p_ref[k]
        num = m[:, :HF]
        den = m[:, HF:HF + H]
        rec = 1.0 / (den + 1e-9)
        recb = _dot(rec, brd)
        zs.append(_elu(num * recb + b_ref[...]))
    z0_ref[...] = zs[0]
    z1_ref[...] = zs[1]
    rows = i * RB + lax.broadcasted_iota(jnp.int32, (RB, 1), 0)
    mask = rows < N
    lane = lax.broadcasted_iota(jnp.int32, (1, HF), 1)
    acc = jnp.zeros((1, HF), jnp.float32)
    for k, z in enumerate(zs):
        t = _dot(jnp.tanh(_dot(z, ws1_ref[...]) + bs1_ref[...]), ws2_ref[...])
        s = jnp.sum(jnp.where(mask, t, 0.0))
        acc = acc + jnp.where(lane == k, s, 0.0)
    sums_ref[pl.ds(i, 1), :] = acc


def _merge(parts, b0, b1, Ws1, bs1, Ws2, Brd):
    full = lambda s: pl.BlockSpec(s, lambda i: (0, 0))
    return pl.pallas_call(
        _merge_body,
        grid=(NTC,),
        in_specs=[
            pl.BlockSpec((2, RB, TBL), lambda i: (0, i, 0)),
            full((1, HF)), full((1, HF)),
            full((HF, HF)), full((1, HF)), full((HF, 1)),
            full((H, HF)),
        ],
        out_specs=[
            pl.BlockSpec((RB, HF), lambda i: (i, 0)),
            pl.BlockSpec((RB, HF), lambda i: (i, 0)),
            pl.BlockSpec((NTC, HF), lambda i: (0, 0)),
        ],
        out_shape=[
            jax.ShapeDtypeStruct((NPAD, HF), jnp.float32),
            jax.ShapeDtypeStruct((NPAD, HF), jnp.float32),
            jax.ShapeDtypeStruct((NTC, HF), jnp.float32),
        ],
    )(parts, b0, b1, Ws1, bs1, Ws2, Brd)


def _final_body(z0_ref, z1_ref, sums_ref, o_ref):
    s = sums_ref[...]
    w0 = jnp.sum(s[:, 0]) / N
    w1 = jnp.sum(s[:, 1]) / N
    m = jnp.maximum(w0, w1)
    e0 = jnp.exp(w0 - m)
    e1 = jnp.exp(w1 - m)
    beta0 = e0 / (e0 + e1)
    beta1 = e1 / (e0 + e1)
    o_ref[...] = beta0 * z0_ref[...] + beta1 * z1_ref[...]


def _final(z0, z1, sums):
    return pl.pallas_call(
        _final_body,
        grid=(NTC,),
        in_specs=[
            pl.BlockSpec((RB, HF), lambda i: (i, 0)),
            pl.BlockSpec((RB, HF), lambda i: (i, 0)),
            pl.BlockSpec((NTC, HF), lambda i: (0, 0)),
        ],
        out_specs=pl.BlockSpec((RB, HF), lambda i: (i, 0)),
        out_shape=jax.ShapeDtypeStruct((NPAD, HF), jnp.float32),
    )(z0, z1, sums)


def _attn_mat(attn_l, attn_r):
    eye = jnp.eye(H, dtype=jnp.float32)
    al = (attn_l[:, :, None] * eye[:, None, :]).reshape(HF, H)
    ar = (attn_r[:, :, None] * eye[:, None, :]).reshape(HF, H)
    return jnp.concatenate([al, ar], axis=1)


def _perm_mat():
    j = jnp.arange(HF)
    g = j // 32
    r = j % 32
    src = 32 * g + (r % 2) * 16 + r // 2
    return (jnp.arange(HF)[:, None] == src[None, :]).astype(jnp.float32)


def _pad_edges(ei):
    src = jnp.concatenate(
        [ei[0], jnp.full((EPAD - E,), N, jnp.int32)]).reshape(NS, NIB, IBLK, C)
    dst = jnp.concatenate(
        [ei[1],
         jnp.full((EPAD - E,), NPAD - 1, jnp.int32)]).reshape(NS, NIB, IBLK, C)
    return src, dst


def kernel(x, edge_index_0, edge_index_1, W_g0, attn_l0, attn_r0, bias0,
           W_g1, attn_l1, attn_r1, bias1, W_s1, b_s1, W_s2):
    x_pad = jnp.zeros((NPAD, D), jnp.float32).at[:N].set(x)
    A0 = _attn_mat(attn_l0, attn_r0)
    A1 = _attn_mat(attn_l1, attn_r1)
    P = _perm_mat()
    eye = jnp.eye(H, dtype=jnp.float32)
    Brd = (eye[:, :, None] * jnp.ones((1, 1, F), jnp.float32)).reshape(H, HF)
    zeros_tbl = jnp.zeros((NPAD, TBL), jnp.float32)

    f0, el0, er0, f1, el1, er1 = _prologue(x_pad, P, W_g0, A0, W_g1, A1)

    src0, dst0 = _pad_edges(edge_index_0)
    src1, dst1 = _pad_edges(edge_index_1)
    src = jnp.stack([src0, src1])
    dst = jnp.stack([dst0, dst1])
    parts = _sc_edge_pass(f0, el0, er0, f1, el1, er1, src, dst, zeros_tbl)

    z0, z1, sums = _merge(parts,
                          bias0.reshape(1, HF), bias1.reshape(1, HF),
                          W_s1, b_s1.reshape(1, HF), W_s2, Brd)
    out = _final(z0, z1, sums)
    return out[:N]


# speedup vs baseline: 1.4263x; 1.0783x over previous
import functools

import jax
import jax.numpy as jnp
from jax import lax
from jax.experimental import pallas as pl
from jax.experimental.pallas import tpu as pltpu
from jax.experimental.pallas import tpu_sc as plsc

N = 10000
E = 320000
D = 128
H = 8
F = 16
HF = H * F
TBL = HF + 2 * H
NPAD = 10240
RB = 1280
NTC = NPAD // RB
NS = 16
C = 64
IBLK = 20
NIB = 16
KCH = NIB * IBLK
EPW = KCH * C
EPAD = NS * EPW
RPS = NPAD // NS

_HIGH = jax.lax.Precision.HIGHEST


def _dot(a, b):
    return jnp.dot(a, b, precision=_HIGH, preferred_element_type=jnp.float32)


def _prologue_body(x_ref, p_ref, w0_ref, a0_ref, w1_ref, a1_ref,
                   f0_ref, el0_ref, er0_ref, f1_ref, el1_ref, er1_ref):
    xb = x_ref[...]
    perm = p_ref[...]
    z8 = jnp.zeros((RB, H), jnp.float32)
    for w_ref, a_ref, f_ref, el_ref, er_ref in (
            (w0_ref, a0_ref, f0_ref, el0_ref, er0_ref),
            (w1_ref, a1_ref, f1_ref, el1_ref, er1_ref)):
        feat = _dot(xb, w_ref[...])
        elr = _dot(feat, a_ref[...])
        f_ref[...] = _dot(feat, perm).astype(jnp.bfloat16)
        el_ref[...] = jnp.concatenate([elr[:, :H], z8], axis=1)
        er_ref[...] = jnp.concatenate([elr[:, H:], z8], axis=1)


def _prologue(x_pad, P, W0, A0, W1, A1):
    full = lambda s: pl.BlockSpec(s, lambda i: (0, 0))
    return pl.pallas_call(
        _prologue_body,
        grid=(NTC,),
        in_specs=[
            pl.BlockSpec((RB, D), lambda i: (i, 0)),
            full((HF, HF)),
            full((D, HF)), full((D, 2 * H)),
            full((D, HF)), full((D, 2 * H)),
        ],
        out_specs=[
            pl.BlockSpec((RB, HF), lambda i: (i, 0)),
            pl.BlockSpec((RB, 16), lambda i: (i, 0)),
            pl.BlockSpec((RB, 16), lambda i: (i, 0)),
            pl.BlockSpec((RB, HF), lambda i: (i, 0)),
            pl.BlockSpec((RB, 16), lambda i: (i, 0)),
            pl.BlockSpec((RB, 16), lambda i: (i, 0)),
        ],
        out_shape=[
            jax.ShapeDtypeStruct((NPAD, HF), jnp.bfloat16),
            jax.ShapeDtypeStruct((NPAD, 16), jnp.float32),
            jax.ShapeDtypeStruct((NPAD, 16), jnp.float32),
            jax.ShapeDtypeStruct((NPAD, HF), jnp.bfloat16),
            jax.ShapeDtypeStruct((NPAD, 16), jnp.float32),
            jax.ShapeDtypeStruct((NPAD, 16), jnp.float32),
        ],
    )(x_pad, P, W0, A0, W1, A1)


def _bcast16(v, h):
    idx = jnp.full((16, 1), h, dtype=jnp.int32)
    dn = lax.GatherDimensionNumbers(
        offset_dims=(), collapsed_slice_dims=(0,), start_index_map=(0,))
    return lax.gather(v, idx, dn, slice_sizes=(1,),
                      mode=lax.GatherScatterMode.PROMISE_IN_BOUNDS)


def _edge_chunk(fb_v, els_v, erd_v, out_v):
    @plsc.parallel_loop(0, C, unroll=2)
    def _edge(i):
        v = els_v[i, :]
        w = erd_v[i, :]
        e = v + w
        e = jnp.maximum(e, 0.2 * e)
        p = jnp.exp(e)
        out_v[i, pl.ds(HF, 16)] = p
        for g in range(4):
            fb = fb_v[i, pl.ds(32 * g, 32)]
            a, b = plsc.unpack(fb, format=plsc.PackFormat.INTERLEAVED,
                               preferred_element_type=jnp.float32)
            out_v[i, pl.ds(32 * g, 16)] = a * _bcast16(p, 2 * g)
            out_v[i, pl.ds(32 * g + 16, 16)] = b * _bcast16(p, 2 * g + 1)


def _run_metapath(feat_hbm, el_hbm, er_hbm, src_hbm, dst_hbm, zeros_hbm, sid,
                  srcs_v, dsts_v, fb_v, els_v, erd_v, out_v,
                  gsems, ssems, isems, acc):
    def _issue(b, sl, dsl, j):
        pltpu.async_copy(feat_hbm.at[srcs_v.at[sl, j]], fb_v.at[b],
                         gsems[3 * b])
        pltpu.async_copy(el_hbm.at[srcs_v.at[sl, j]], els_v.at[b],
                         gsems[3 * b + 1])
        pltpu.async_copy(er_hbm.at[dsts_v.at[dsl, j]], erd_v.at[b],
                         gsems[3 * b + 2])

    def _await_gather(b):
        pltpu.make_async_copy(feat_hbm.at[srcs_v.at[0, 0]], fb_v.at[b],
                              gsems[3 * b]).wait()
        pltpu.make_async_copy(el_hbm.at[srcs_v.at[0, 0]], els_v.at[b],
                              gsems[3 * b + 1]).wait()
        pltpu.make_async_copy(er_hbm.at[dsts_v.at[0, 0]], erd_v.at[b],
                              gsems[3 * b + 2]).wait()

    def _issue_scatter(b, dsl, j):
        pltpu.async_copy(out_v.at[b], acc.at[dsts_v.at[dsl, j]], ssems[b],
                         add=True)

    def _await_scatter(b):
        pltpu.make_async_copy(out_v.at[b], acc.at[dsts_v.at[0, 0]],
                              ssems[b]).wait()

    pltpu.sync_copy(src_hbm.at[sid, 0], srcs_v.at[0])
    pltpu.sync_copy(dst_hbm.at[sid, 0], dsts_v.at[0])

    for b in range(2):
        pltpu.sync_copy(zeros_hbm.at[pl.ds(0, C)], out_v.at[b])
        _issue_scatter(b, 0, b)
    _issue(0, 0, 0, 0)
    _issue(1, 0, 0, 1)

    @pl.loop(0, NIB)
    def _block(ib):
        sl = ib % 2
        dsl = ib % 3
        nsl = (ib + 1) % 2
        ndsl = (ib + 1) % 3

        @pl.when(ib + 1 < NIB)
        def _():
            pltpu.async_copy(src_hbm.at[sid, ib + 1], srcs_v.at[nsl],
                             isems[0])
            pltpu.async_copy(dst_hbm.at[sid, ib + 1], dsts_v.at[ndsl],
                             isems[1])

        @pl.loop(0, IBLK, step=2)
        def _chunk(l):
            for b in range(2):
                ll = l + b
                _await_gather(b)
                _await_scatter(b)
                _edge_chunk(fb_v.at[b], els_v.at[b], erd_v.at[b], out_v.at[b])
                _issue_scatter(b, dsl, ll)

                @pl.when(ll + 2 < IBLK)
                def _():
                    _issue(b, sl, dsl, ll + 2)

        @pl.when(ib + 1 < NIB)
        def _():
            pltpu.make_async_copy(src_hbm.at[sid, 0], srcs_v.at[0],
                                  isems[0]).wait()
            pltpu.make_async_copy(dst_hbm.at[sid, 0], dsts_v.at[0],
                                  isems[1]).wait()
            _issue(0, nsl, ndsl, 0)
            _issue(1, nsl, ndsl, 1)

    _await_scatter(0)
    _await_scatter(1)


def _sc_edge_body(f0_hbm, el0_hbm, er0_hbm, f1_hbm, el1_hbm, er1_hbm,
                  src0_hbm, dst0_hbm, src1_hbm, dst1_hbm, zeros_hbm, out_hbm,
                  srcs_v, dsts_v, fb_v, els_v, erd_v, out_v,
                  g0, g1, g2, g3, g4, g5, s0, s1, i0, i1, acc):
    cid = lax.axis_index("c")
    sid = lax.axis_index("s")
    r0 = sid * RPS
    pltpu.sync_copy(zeros_hbm.at[pl.ds(r0, RPS)], acc.at[pl.ds(r0, RPS)])
    plsc.subcore_barrier()

    gsems = (g0, g1, g2, g3, g4, g5)
    ssems = (s0, s1)
    isems = (i0, i1)

    @pl.when(cid == 0)
    def _():
        _run_metapath(f0_hbm, el0_hbm, er0_hbm, src0_hbm, dst0_hbm,
                      zeros_hbm, sid, srcs_v, dsts_v, fb_v, els_v, erd_v,
                      out_v, gsems, ssems, isems, acc)

    @pl.when(cid == 1)
    def _():
        _run_metapath(f1_hbm, el1_hbm, er1_hbm, src1_hbm, dst1_hbm,
                      zeros_hbm, sid, srcs_v, dsts_v, fb_v, els_v, erd_v,
                      out_v, gsems, ssems, isems, acc)

    plsc.subcore_barrier()
    pltpu.sync_copy(acc.at[pl.ds(r0, RPS)], out_hbm.at[cid, pl.ds(r0, RPS)])


_sc_edge_pass = pl.kernel(
    _sc_edge_body,
    out_type=jax.ShapeDtypeStruct((2, NPAD, TBL), jnp.float32),
    mesh=plsc.VectorSubcoreMesh(core_axis_name="c", subcore_axis_name="s"),
    compiler_params=pltpu.CompilerParams(use_tc_tiling_on_sc=False,
                                         needs_layout_passes=False),
    scratch_types=[
        pltpu.VMEM((2, IBLK, C), jnp.int32),
        pltpu.VMEM((3, IBLK, C), jnp.int32),
        pltpu.VMEM((2, C, HF), jnp.bfloat16),
        pltpu.VMEM((2, C, 16), jnp.float32),
        pltpu.VMEM((2, C, 16), jnp.float32),
        pltpu.VMEM((2, C, TBL), jnp.float32),
        pltpu.SemaphoreType.DMA,
        pltpu.SemaphoreType.DMA,
        pltpu.SemaphoreType.DMA,
        pltpu.SemaphoreType.DMA,
        pltpu.SemaphoreType.DMA,
        pltpu.SemaphoreType.DMA,
        pltpu.SemaphoreType.DMA,
        pltpu.SemaphoreType.DMA,
        pltpu.SemaphoreType.DMA,
        pltpu.SemaphoreType.DMA,
        pltpu.VMEM_SHARED((NPAD, TBL), jnp.float32),
    ],
)


def _elu(x):
    return jnp.where(x > 0, x, jnp.exp(jnp.minimum(x, 0.0)) - 1.0)


def _merge_body(p_ref, b0_ref, b1_ref, ws1_ref, bs1_ref, ws2_ref,
                brd_ref, z0_ref, z1_ref, sums_ref):
    i = pl.program_id(0)
    brd = brd_ref[...]
    zs = []
    for k, b_ref in ((0, b0_ref), (1, b1_ref)):
        m = p_ref[k]
        num = m[:, :HF]
        den = m[:, HF:HF + H]
        rec = 1.0 / (den + 1e-9)
        recb = _dot(rec, brd)
        zs.append(_elu(num * recb + b_ref[...]))
    z0_ref[...] = zs[0]
    z1_ref[...] = zs[1]
    rows = i * RB + lax.broadcasted_iota(jnp.int32, (RB, 1), 0)
    mask = rows < N
    lane = lax.broadcasted_iota(jnp.int32, (1, HF), 1)
    acc = jnp.zeros((1, HF), jnp.float32)
    for k, z in enumerate(zs):
        t = _dot(jnp.tanh(_dot(z, ws1_ref[...]) + bs1_ref[...]), ws2_ref[...])
        s = jnp.sum(jnp.where(mask, t, 0.0))
        acc = acc + jnp.where(lane == k, s, 0.0)
    sums_ref[pl.ds(i, 1), :] = acc


def _merge(parts, b0, b1, Ws1, bs1, Ws2, Brd):
    full = lambda s: pl.BlockSpec(s, lambda i: (0, 0))
    return pl.pallas_call(
        _merge_body,
        grid=(NTC,),
        in_specs=[
            pl.BlockSpec((2, RB, TBL), lambda i: (0, i, 0)),
            full((1, HF)), full((1, HF)),
            full((HF, HF)), full((1, HF)), full((HF, 1)),
            full((H, HF)),
        ],
        out_specs=[
            pl.BlockSpec((RB, HF), lambda i: (i, 0)),
            pl.BlockSpec((RB, HF), lambda i: (i, 0)),
            pl.BlockSpec((NTC, HF), lambda i: (0, 0)),
        ],
        out_shape=[
            jax.ShapeDtypeStruct((NPAD, HF), jnp.float32),
            jax.ShapeDtypeStruct((NPAD, HF), jnp.float32),
            jax.ShapeDtypeStruct((NTC, HF), jnp.float32),
        ],
    )(parts, b0, b1, Ws1, bs1, Ws2, Brd)


FB = 2000


def _final_body(z0_ref, z1_ref, sums_ref, o_ref):
    s = sums_ref[...]
    w0 = jnp.sum(s[:, 0]) / N
    w1 = jnp.sum(s[:, 1]) / N
    m = jnp.maximum(w0, w1)
    e0 = jnp.exp(w0 - m)
    e1 = jnp.exp(w1 - m)
    beta0 = e0 / (e0 + e1)
    beta1 = e1 / (e0 + e1)
    o_ref[...] = beta0 * z0_ref[...] + beta1 * z1_ref[...]


def _final(z0, z1, sums):
    return pl.pallas_call(
        _final_body,
        grid=(N // FB,),
        in_specs=[
            pl.BlockSpec((FB, HF), lambda i: (i, 0)),
            pl.BlockSpec((FB, HF), lambda i: (i, 0)),
            pl.BlockSpec((NTC, HF), lambda i: (0, 0)),
        ],
        out_specs=pl.BlockSpec((FB, HF), lambda i: (i, 0)),
        out_shape=jax.ShapeDtypeStruct((N, HF), jnp.float32),
    )(z0, z1, sums)


def _attn_mat(attn_l, attn_r):
    eye = jnp.eye(H, dtype=jnp.float32)
    al = (attn_l[:, :, None] * eye[:, None, :]).reshape(HF, H)
    ar = (attn_r[:, :, None] * eye[:, None, :]).reshape(HF, H)
    return jnp.concatenate([al, ar], axis=1)


def _perm_mat():
    j = jnp.arange(HF)
    g = j // 32
    r = j % 32
    src = 32 * g + (r % 2) * 16 + r // 2
    return (jnp.arange(HF)[:, None] == src[None, :]).astype(jnp.float32)


def _pad_edges(ei):
    src = jnp.concatenate(
        [ei[0], jnp.full((EPAD - E,), N, jnp.int32)]).reshape(NS, NIB, IBLK, C)
    dst = jnp.concatenate(
        [ei[1],
         jnp.full((EPAD - E,), NPAD - 1, jnp.int32)]).reshape(NS, NIB, IBLK, C)
    return src, dst


def kernel(x, edge_index_0, edge_index_1, W_g0, attn_l0, attn_r0, bias0,
           W_g1, attn_l1, attn_r1, bias1, W_s1, b_s1, W_s2):
    x_pad = jnp.zeros((NPAD, D), jnp.float32).at[:N].set(x)
    A0 = _attn_mat(attn_l0, attn_r0)
    A1 = _attn_mat(attn_l1, attn_r1)
    P = _perm_mat()
    eye = jnp.eye(H, dtype=jnp.float32)
    Brd = (eye[:, :, None] * jnp.ones((1, 1, F), jnp.float32)).reshape(H, HF)
    zeros_tbl = jnp.zeros((NPAD, TBL), jnp.float32)

    f0, el0, er0, f1, el1, er1 = _prologue(x_pad, P, W_g0, A0, W_g1, A1)

    src0, dst0 = _pad_edges(edge_index_0)
    src1, dst1 = _pad_edges(edge_index_1)
    parts = _sc_edge_pass(f0, el0, er0, f1, el1, er1,
                          src0, dst0, src1, dst1, zeros_tbl)

    z0, z1, sums = _merge(parts,
                          bias0.reshape(1, HF), bias1.reshape(1, HF),
                          W_s1, b_s1.reshape(1, HF), W_s2, Brd)
    return _final(z0, z1, sums)
